# baseline (speedup 1.0000x reference)
import functools

import jax
import jax.numpy as jnp
from jax import lax
from jax.experimental import pallas as pl
from jax.experimental.pallas import tpu as pltpu
from jax.experimental.pallas import tpu_sc as plsc

B = 1024
N = 100000
Q = 0.7
K = 0.5

RB = 32
NRB = B // RB

NC, NS, L = 2, 16, 16
NW = NC * NS
BPW = B // NW


@functools.lru_cache(maxsize=None)
def _sc_gather_build():
    mesh = plsc.VectorSubcoreMesh(core_axis_name="c", subcore_axis_name="s")

    @functools.partial(
        pl.kernel,
        mesh=mesh,
        out_type=[
            jax.ShapeDtypeStruct((B,), jnp.int32),
            jax.ShapeDtypeStruct((B,), jnp.float32),
        ],
        scratch_types=[
            pltpu.VMEM((BPW,), jnp.int32),
            pltpu.VMEM((BPW,), jnp.int32),
            pltpu.VMEM((BPW,), jnp.float32),
            pltpu.SemaphoreType.DMA,
        ],
    )
    def sc_gather(idx_hbm, lab_hbm, w_hbm, y_out, wb_out,
                  idx_v, y_v, w_v, sem):
        wid = lax.axis_index("s") * NC + lax.axis_index("c")
        base = wid * BPW
        pltpu.sync_copy(idx_hbm.at[pl.ds(base, BPW)], idx_v)
        pltpu.async_copy(lab_hbm.at[idx_v], y_v, sem).wait()
        pltpu.async_copy(w_hbm.at[idx_v], w_v, sem).wait()
        pltpu.sync_copy(y_v, y_out.at[pl.ds(base, BPW)])
        pltpu.sync_copy(w_v, wb_out.at[pl.ds(base, BPW)])

    return sc_gather


def _tc_body(xb, labb, y, idxb, wb, out, a_sum, w_sum):
    i = pl.program_id(0)

    @pl.when(i == 0)
    def _init():
        a_sum[0, 0] = 0.0
        w_sum[0, 0] = 0.0

    e = jnp.exp(xb[...])
    col = lax.broadcasted_iota(jnp.int32, (1, N), 1)
    e = jnp.where(col == idxb[...], 0.0, e)
    m = labb[...] == y[...]
    z = jnp.sum(e, axis=1, keepdims=True)
    p = jnp.sum(jnp.where(m, e, 0.0), axis=1, keepdims=True)
    prob = p / z
    a = (1.0 - prob ** Q) / Q
    a_sum[0, 0] += jnp.sum(a)
    w_sum[0, 0] += jnp.sum(wb[...])

    @pl.when(i == NRB - 1)
    def _fin():
        mean_w = w_sum[0, 0] * (1.0 / B)
        out[0, 0] = (a_sum[0, 0] * (1.0 / B)) * mean_w \
            - ((1.0 - K ** Q) / Q) * mean_w


_tc_call = pl.pallas_call(
    _tc_body,
    grid=(NRB,),
    in_specs=[
        pl.BlockSpec((RB, N), lambda i: (i, 0)),
        pl.BlockSpec((1, N), lambda i: (0, 0)),
        pl.BlockSpec((RB, 1), lambda i: (i, 0)),
        pl.BlockSpec((RB, 1), lambda i: (i, 0)),
        pl.BlockSpec((RB, 1), lambda i: (i, 0)),
    ],
    out_specs=pl.BlockSpec(memory_space=pltpu.SMEM),
    out_shape=jax.ShapeDtypeStruct((1, 1), jnp.float32),
    scratch_shapes=[
        pltpu.SMEM((1, 1), jnp.float32),
        pltpu.SMEM((1, 1), jnp.float32),
    ],
    compiler_params=pltpu.CompilerParams(
        dimension_semantics=("arbitrary",),
    ),
)


def kernel(x, indexes, labels, weights):
    idx = indexes.astype(jnp.int32)
    lab = labels.astype(jnp.int32)
    y, wb = _sc_gather_build()(idx, lab, weights.reshape(-1))
    loss = _tc_call(x, lab.reshape(1, N), y.reshape(B, 1),
                    idx.reshape(B, 1), wb.reshape(B, 1))
    return loss[0, 0]
